# baseline (speedup 1.0000x reference)
import jax
import jax.numpy as jnp
from jax import lax
from jax.experimental import pallas as pl
from jax.experimental.pallas import tpu as pltpu
from jax.experimental.pallas import tpu_sc as plsc

BATCH = 16384
EMBED = 32
BLOCK = 128
PACK = BLOCK // EMBED
NUM_CORES = 2
NUM_SUBCORES = 16
NUM_WORKERS = NUM_CORES * NUM_SUBCORES
B_PER_W = BATCH // NUM_WORKERS
GCHUNK = 128
N_GCHUNK = B_PER_W // GCHUNK
IDX_ROWS = BATCH // GCHUNK


def _gather_body(uidx_hbm, iidx_hbm, ut_hbm, it_hbm, uout_hbm, iout_hbm,
                 idx_v, rows_v, sem):
    wid = lax.axis_index("s") * NUM_CORES + lax.axis_index("c")
    base = wid * B_PER_W
    irow = wid * N_GCHUNK

    def run(idx_hbm, t_hbm, out_hbm):
        pltpu.sync_copy(idx_hbm.at[pl.ds(irow, N_GCHUNK), :], idx_v)
        for j in range(N_GCHUNK):
            pltpu.async_copy(t_hbm.at[idx_v.at[j]],
                             rows_v.at[pl.ds(j * GCHUNK, GCHUNK), :], sem)
        for j in range(N_GCHUNK):
            pltpu.make_async_copy(t_hbm.at[idx_v.at[j]],
                                  rows_v.at[pl.ds(j * GCHUNK, GCHUNK), :],
                                  sem).wait()
        pltpu.sync_copy(rows_v, out_hbm.at[pl.ds(base, B_PER_W), :])

    run(uidx_hbm, ut_hbm, uout_hbm)
    run(iidx_hbm, it_hbm, iout_hbm)


def _sc_gather(ubidx, ibidx, ut128, it128):
    mesh = plsc.VectorSubcoreMesh(core_axis_name="c", subcore_axis_name="s")
    k = pl.kernel(
        _gather_body,
        out_type=[
            jax.ShapeDtypeStruct((BATCH, BLOCK), jnp.float32),
            jax.ShapeDtypeStruct((BATCH, BLOCK), jnp.float32),
        ],
        mesh=mesh,
        scratch_types=[
            pltpu.VMEM((N_GCHUNK, GCHUNK), jnp.int32),
            pltpu.VMEM((B_PER_W, BLOCK), jnp.float32),
            pltpu.SemaphoreType.DMA,
        ],
    )
    return k(ubidx, ibidx, ut128, it128)


def _select(blocks, sub):
    out = jnp.zeros((blocks.shape[0], EMBED), jnp.float32)
    for k in range(PACK):
        m = (sub == k).astype(jnp.float32)
        out = out + m * blocks[:, k * EMBED:(k + 1) * EMBED]
    return out


def _mlp_body(ub_ref, ib_ref, us_ref, is_ref, w1u_ref, w1v_ref, b1_ref,
              w2_ref, b2_ref, w3_ref, b3_ref, wo_ref, bo_ref, out_ref):
    u = _select(ub_ref[...], us_ref[...])
    v = _select(ib_ref[...], is_ref[...])
    x = (jnp.dot(u, w1u_ref[...], preferred_element_type=jnp.float32)
         + jnp.dot(v, w1v_ref[...], preferred_element_type=jnp.float32)
         + b1_ref[...])
    x = jnp.maximum(x, 0.0)
    x = jnp.dot(x, w2_ref[...], preferred_element_type=jnp.float32) + b2_ref[...]
    x = jnp.maximum(x, 0.0)
    x = jnp.dot(x, w3_ref[...], preferred_element_type=jnp.float32) + b3_ref[...]
    x = jnp.maximum(x, 0.0)
    y = jnp.dot(x, wo_ref[...], preferred_element_type=jnp.float32) + bo_ref[...]
    out_ref[...] = 1.0 / (1.0 + jnp.exp(-y))


def _tc_mlp(ublocks, iblocks, usub, isub, W1, b1, W2, b2, W3, b3, Wo, bo,
            *, tile=2048):
    grid = BATCH // tile
    full = lambda shape: pl.BlockSpec(shape, lambda i: (0, 0))
    return pl.pallas_call(
        _mlp_body,
        grid=(grid,),
        in_specs=[
            pl.BlockSpec((tile, BLOCK), lambda i: (i, 0)),
            pl.BlockSpec((tile, BLOCK), lambda i: (i, 0)),
            pl.BlockSpec((tile, 1), lambda i: (i, 0)),
            pl.BlockSpec((tile, 1), lambda i: (i, 0)),
            full((EMBED, 128)),
            full((EMBED, 128)),
            full((1, 128)),
            full((128, 64)),
            full((1, 64)),
            full((64, 32)),
            full((1, 32)),
            full((32, 1)),
            full((1, 1)),
        ],
        out_specs=pl.BlockSpec((tile, 1), lambda i: (i, 0)),
        out_shape=jax.ShapeDtypeStruct((BATCH, 1), jnp.float32),
    )(ublocks, iblocks, usub, isub,
      W1[:EMBED], W1[EMBED:], b1.reshape(1, -1), W2, b2.reshape(1, -1),
      W3, b3.reshape(1, -1), Wo, bo.reshape(1, -1))


def kernel(user_indices, item_indices, user_table, item_table,
           W1, b1, W2, b2, W3, b3, Wo, bo):
    uidx = user_indices.astype(jnp.int32)
    iidx = item_indices.astype(jnp.int32)
    ubidx = (uidx // PACK).reshape(IDX_ROWS, GCHUNK)
    ibidx = (iidx // PACK).reshape(IDX_ROWS, GCHUNK)
    ut128 = user_table.reshape(-1, BLOCK)
    it128 = item_table.reshape(-1, BLOCK)
    ublocks, iblocks = _sc_gather(ubidx, ibidx, ut128, it128)
    usub = (uidx % PACK).reshape(BATCH, 1)
    isub = (iidx % PACK).reshape(BATCH, 1)
    return _tc_mlp(ublocks, iblocks, usub, isub,
                   W1, b1, W2, b2, W3, b3, Wo, bo)


# speedup vs baseline: 1.4435x; 1.4435x over previous
import jax
import jax.numpy as jnp
from jax import lax
from jax.experimental import pallas as pl
from jax.experimental.pallas import tpu as pltpu
from jax.experimental.pallas import tpu_sc as plsc

BATCH = 16384
EMBED = 32
ROWS = 1_000_000
BLOCK = 128
PACK = BLOCK // EMBED
CHUNK = 512
TPG = 8
PGRID = -(-ROWS // (CHUNK * TPG))
PROWS = PGRID * TPG * 128
NUM_CORES = 2
NUM_SUBCORES = 16
NUM_WORKERS = NUM_CORES * NUM_SUBCORES
B_PER_W = BATCH // NUM_WORKERS
GCHUNK = 128
N_GCHUNK = B_PER_W // GCHUNK
IDX_ROWS = BATCH // GCHUNK


def _pack_body(x_ref, out_ref):
    for tt in range(TPG):
        for c in range(PACK):
            src = x_ref[:, CHUNK * tt + 128 * c:CHUNK * tt + 128 * c + 128]
            out_ref[128 * tt:128 * tt + 128,
                    EMBED * c:EMBED * c + EMBED] = src.T


def _tc_pack(table_t):
    return pl.pallas_call(
        _pack_body,
        grid=(PGRID,),
        in_specs=[pl.BlockSpec((EMBED, CHUNK * TPG), lambda i: (0, i))],
        out_specs=pl.BlockSpec((TPG * 128, BLOCK), lambda i: (i, 0)),
        out_shape=jax.ShapeDtypeStruct((PROWS, BLOCK), jnp.float32),
    )(table_t)


def _gather_body(idx_hbm, t_hbm, out_hbm, idx_v, rows_v, sem):
    wid = lax.axis_index("s") * NUM_CORES + lax.axis_index("c")
    base = wid * B_PER_W
    irow = wid * N_GCHUNK
    pltpu.sync_copy(idx_hbm.at[pl.ds(irow, N_GCHUNK), :], idx_v)
    for j in range(N_GCHUNK):
        pltpu.async_copy(t_hbm.at[idx_v.at[j]],
                         rows_v.at[pl.ds(j * GCHUNK, GCHUNK), :], sem)
    for j in range(N_GCHUNK):
        pltpu.make_async_copy(t_hbm.at[idx_v.at[j]],
                              rows_v.at[pl.ds(j * GCHUNK, GCHUNK), :],
                              sem).wait()
    pltpu.sync_copy(rows_v, out_hbm.at[pl.ds(base, B_PER_W), :])


def _sc_gather(bidx, packed):
    mesh = plsc.VectorSubcoreMesh(core_axis_name="c", subcore_axis_name="s")
    k = pl.kernel(
        _gather_body,
        out_type=jax.ShapeDtypeStruct((BATCH, BLOCK), jnp.float32),
        mesh=mesh,
        scratch_types=[
            pltpu.VMEM((N_GCHUNK, GCHUNK), jnp.int32),
            pltpu.VMEM((B_PER_W, BLOCK), jnp.float32),
            pltpu.SemaphoreType.DMA,
        ],
    )
    return k(bidx, packed)


def _select(blocks, sub):
    out = jnp.zeros((blocks.shape[0], EMBED), jnp.float32)
    for k in range(PACK):
        m = (sub == k).astype(jnp.float32)
        out = out + m * blocks[:, k * EMBED:(k + 1) * EMBED]
    return out


def _mlp_body(ub_ref, ib_ref, us_ref, is_ref, w1u_ref, w1v_ref, b1_ref,
              w2_ref, b2_ref, w3_ref, b3_ref, wo_ref, bo_ref, out_ref):
    u = _select(ub_ref[...], us_ref[...])
    v = _select(ib_ref[...], is_ref[...])
    x = (jnp.dot(u, w1u_ref[...], preferred_element_type=jnp.float32)
         + jnp.dot(v, w1v_ref[...], preferred_element_type=jnp.float32)
         + b1_ref[...])
    x = jnp.maximum(x, 0.0)
    x = jnp.dot(x, w2_ref[...], preferred_element_type=jnp.float32) + b2_ref[...]
    x = jnp.maximum(x, 0.0)
    x = jnp.dot(x, w3_ref[...], preferred_element_type=jnp.float32) + b3_ref[...]
    x = jnp.maximum(x, 0.0)
    y = jnp.dot(x, wo_ref[...], preferred_element_type=jnp.float32) + bo_ref[...]
    out_ref[...] = 1.0 / (1.0 + jnp.exp(-y))


def _tc_mlp(ublocks, iblocks, usub, isub, W1, b1, W2, b2, W3, b3, Wo, bo,
            *, tile=2048):
    grid = BATCH // tile
    full = lambda shape: pl.BlockSpec(shape, lambda i: (0, 0))
    return pl.pallas_call(
        _mlp_body,
        grid=(grid,),
        in_specs=[
            pl.BlockSpec((tile, BLOCK), lambda i: (i, 0)),
            pl.BlockSpec((tile, BLOCK), lambda i: (i, 0)),
            pl.BlockSpec((tile, 1), lambda i: (i, 0)),
            pl.BlockSpec((tile, 1), lambda i: (i, 0)),
            full((EMBED, 128)),
            full((EMBED, 128)),
            full((1, 128)),
            full((128, 64)),
            full((1, 64)),
            full((64, 32)),
            full((1, 32)),
            full((32, 1)),
            full((1, 1)),
        ],
        out_specs=pl.BlockSpec((tile, 1), lambda i: (i, 0)),
        out_shape=jax.ShapeDtypeStruct((BATCH, 1), jnp.float32),
    )(ublocks, iblocks, usub, isub,
      W1[:EMBED], W1[EMBED:], b1.reshape(1, -1), W2, b2.reshape(1, -1),
      W3, b3.reshape(1, -1), Wo, bo.reshape(1, -1))


def kernel(user_indices, item_indices, user_table, item_table,
           W1, b1, W2, b2, W3, b3, Wo, bo):
    uidx = user_indices.astype(jnp.int32)
    iidx = item_indices.astype(jnp.int32)
    ubidx = (128 * (uidx // CHUNK) + uidx % 128).reshape(IDX_ROWS, GCHUNK)
    ibidx = (128 * (iidx // CHUNK) + iidx % 128).reshape(IDX_ROWS, GCHUNK)
    upacked = _tc_pack(user_table.T)
    ublocks = _sc_gather(ubidx, upacked)
    ipacked = _tc_pack(item_table.T)
    iblocks = _sc_gather(ibidx, ipacked)
    usub = ((uidx // 128) % PACK).reshape(BATCH, 1)
    isub = ((iidx // 128) % PACK).reshape(BATCH, 1)
    return _tc_mlp(ublocks, iblocks, usub, isub,
                   W1, b1, W2, b2, W3, b3, Wo, bo)


# speedup vs baseline: 2.0496x; 1.4198x over previous
import jax
import jax.numpy as jnp
from jax import lax
from jax.experimental import pallas as pl
from jax.experimental.pallas import tpu as pltpu
from jax.experimental.pallas import tpu_sc as plsc

BATCH = 16384
EMBED = 32
ROWS = 1_000_000
BLOCK = 128
PACK = BLOCK // EMBED
CHUNK = 512
TPG = 8
PGRID = -(-ROWS // (CHUNK * TPG))
PROWS = PGRID * TPG * 128
NUM_CORES = 2
NUM_SUBCORES = 16
NUM_WORKERS = NUM_CORES * NUM_SUBCORES
B_PER_W = BATCH // NUM_WORKERS
GCHUNK = 128
N_GCHUNK = B_PER_W // GCHUNK
IDX_ROWS = BATCH // GCHUNK


def _pack_body(x_ref, out_ref):
    for tt in range(TPG):
        x4 = jnp.concatenate(
            [x_ref[:, CHUNK * tt + 128 * c:CHUNK * tt + 128 * c + 128]
             for c in range(PACK)], axis=0)
        out_ref[128 * tt:128 * tt + 128, :] = x4.T


def _tc_pack(table_t):
    return pl.pallas_call(
        _pack_body,
        grid=(PGRID,),
        in_specs=[pl.BlockSpec((EMBED, CHUNK * TPG), lambda i: (0, i))],
        out_specs=pl.BlockSpec((TPG * 128, BLOCK), lambda i: (i, 0)),
        out_shape=jax.ShapeDtypeStruct((PROWS, BLOCK), jnp.float32),
        compiler_params=pltpu.CompilerParams(
            dimension_semantics=("parallel",)),
    )(table_t)


def _gather_body(idx_hbm, t_hbm, out_hbm, idx_v, rows_v, sem):
    wid = lax.axis_index("s") * NUM_CORES + lax.axis_index("c")
    base = wid * B_PER_W
    irow = wid * N_GCHUNK
    pltpu.sync_copy(idx_hbm.at[pl.ds(irow, N_GCHUNK), :], idx_v)
    for j in range(N_GCHUNK):
        pltpu.async_copy(t_hbm.at[idx_v.at[j]],
                         rows_v.at[pl.ds(j * GCHUNK, GCHUNK), :], sem)
    for j in range(N_GCHUNK):
        pltpu.make_async_copy(t_hbm.at[idx_v.at[j]],
                              rows_v.at[pl.ds(j * GCHUNK, GCHUNK), :],
                              sem).wait()
    pltpu.sync_copy(rows_v, out_hbm.at[pl.ds(base, B_PER_W), :])


def _sc_gather(bidx, packed):
    mesh = plsc.VectorSubcoreMesh(core_axis_name="c", subcore_axis_name="s")
    k = pl.kernel(
        _gather_body,
        out_type=jax.ShapeDtypeStruct((BATCH, BLOCK), jnp.float32),
        mesh=mesh,
        scratch_types=[
            pltpu.VMEM((N_GCHUNK, GCHUNK), jnp.int32),
            pltpu.VMEM((B_PER_W, BLOCK), jnp.float32),
            pltpu.SemaphoreType.DMA,
        ],
    )
    return k(bidx, packed)


def _select(blocks, sub):
    out = jnp.zeros((blocks.shape[0], EMBED), jnp.float32)
    for k in range(PACK):
        m = (sub == k).astype(jnp.float32)
        out = out + m * blocks[:, k * EMBED:(k + 1) * EMBED]
    return out


def _mlp_body(ub_ref, ib_ref, us_ref, is_ref, w1u_ref, w1v_ref, b1_ref,
              w2_ref, b2_ref, w3_ref, b3_ref, wo_ref, bo_ref, out_ref):
    u = _select(ub_ref[...], us_ref[...])
    v = _select(ib_ref[...], is_ref[...])
    x = (jnp.dot(u, w1u_ref[...], preferred_element_type=jnp.float32)
         + jnp.dot(v, w1v_ref[...], preferred_element_type=jnp.float32)
         + b1_ref[...])
    x = jnp.maximum(x, 0.0)
    x = jnp.dot(x, w2_ref[...], preferred_element_type=jnp.float32) + b2_ref[...]
    x = jnp.maximum(x, 0.0)
    x = jnp.dot(x, w3_ref[...], preferred_element_type=jnp.float32) + b3_ref[...]
    x = jnp.maximum(x, 0.0)
    y = jnp.dot(x, wo_ref[...], preferred_element_type=jnp.float32) + bo_ref[...]
    out_ref[...] = 1.0 / (1.0 + jnp.exp(-y))


def _tc_mlp(ublocks, iblocks, usub, isub, W1, b1, W2, b2, W3, b3, Wo, bo,
            *, tile=2048):
    grid = BATCH // tile
    full = lambda shape: pl.BlockSpec(shape, lambda i: (0, 0))
    return pl.pallas_call(
        _mlp_body,
        grid=(grid,),
        in_specs=[
            pl.BlockSpec((tile, BLOCK), lambda i: (i, 0)),
            pl.BlockSpec((tile, BLOCK), lambda i: (i, 0)),
            pl.BlockSpec((tile, 1), lambda i: (i, 0)),
            pl.BlockSpec((tile, 1), lambda i: (i, 0)),
            full((EMBED, 128)),
            full((EMBED, 128)),
            full((1, 128)),
            full((128, 64)),
            full((1, 64)),
            full((64, 32)),
            full((1, 32)),
            full((32, 1)),
            full((1, 1)),
        ],
        out_specs=pl.BlockSpec((tile, 1), lambda i: (i, 0)),
        out_shape=jax.ShapeDtypeStruct((BATCH, 1), jnp.float32),
        compiler_params=pltpu.CompilerParams(
            dimension_semantics=("parallel",)),
    )(ublocks, iblocks, usub, isub,
      W1[:EMBED], W1[EMBED:], b1.reshape(1, -1), W2, b2.reshape(1, -1),
      W3, b3.reshape(1, -1), Wo, bo.reshape(1, -1))


def kernel(user_indices, item_indices, user_table, item_table,
           W1, b1, W2, b2, W3, b3, Wo, bo):
    uidx = user_indices.astype(jnp.int32)
    iidx = item_indices.astype(jnp.int32)
    ubidx = (128 * (uidx // CHUNK) + uidx % 128).reshape(IDX_ROWS, GCHUNK)
    ibidx = (128 * (iidx // CHUNK) + iidx % 128).reshape(IDX_ROWS, GCHUNK)
    upacked = _tc_pack(user_table.T)
    ublocks = _sc_gather(ubidx, upacked)
    ipacked = _tc_pack(item_table.T)
    iblocks = _sc_gather(ibidx, ipacked)
    usub = ((uidx // 128) % PACK).reshape(BATCH, 1)
    isub = ((iidx // 128) % PACK).reshape(BATCH, 1)
    return _tc_mlp(ublocks, iblocks, usub, isub,
                   W1, b1, W2, b2, W3, b3, Wo, bo)


# speedup vs baseline: 2.1140x; 1.0314x over previous
import jax
import jax.numpy as jnp
from jax import lax
from jax.experimental import pallas as pl
from jax.experimental.pallas import tpu as pltpu
from jax.experimental.pallas import tpu_sc as plsc

BATCH = 16384
EMBED = 32
ROWS = 1_000_000
BLOCK = 128
PACK = BLOCK // EMBED
CHUNK = 512
TPG = 8
PGRID = -(-ROWS // (CHUNK * TPG))
PROWS = PGRID * TPG * 128
NUM_CORES = 2
NUM_SUBCORES = 16
NUM_WORKERS = NUM_CORES * NUM_SUBCORES
B_PER_W = BATCH // NUM_WORKERS
GCHUNK = 128
N_GCHUNK = B_PER_W // GCHUNK
IDX_ROWS = BATCH // GCHUNK


def _pack_body(x_ref, out_ref):
    for tt in range(TPG):
        x4 = jnp.concatenate(
            [x_ref[:, CHUNK * tt + 128 * c:CHUNK * tt + 128 * c + 128]
             for c in range(PACK)], axis=0)
        out_ref[128 * tt:128 * tt + 128, :] = x4.T


def _tc_pack(table_t):
    return pl.pallas_call(
        _pack_body,
        grid=(PGRID,),
        in_specs=[pl.BlockSpec((EMBED, CHUNK * TPG), lambda i: (0, i))],
        out_specs=pl.BlockSpec((TPG * 128, BLOCK), lambda i: (i, 0)),
        out_shape=jax.ShapeDtypeStruct((PROWS, BLOCK), jnp.float32),
        compiler_params=pltpu.CompilerParams(
            dimension_semantics=("parallel",)),
    )(table_t)


def _gather_body(idx_hbm, t_hbm, out_hbm, idx_v, rows_v, sem):
    wid = lax.axis_index("s") * NUM_CORES + lax.axis_index("c")
    base = wid * B_PER_W
    irow = wid * N_GCHUNK
    pltpu.sync_copy(idx_hbm.at[pl.ds(irow, N_GCHUNK), :], idx_v)
    for j in range(N_GCHUNK):
        pltpu.async_copy(t_hbm.at[idx_v.at[j]],
                         rows_v.at[pl.ds(j * GCHUNK, GCHUNK), :], sem)
    for j in range(N_GCHUNK):
        pltpu.make_async_copy(t_hbm.at[idx_v.at[j]],
                              rows_v.at[pl.ds(j * GCHUNK, GCHUNK), :],
                              sem).wait()
    pltpu.sync_copy(rows_v, out_hbm.at[pl.ds(base, B_PER_W), :])


def _sc_gather(bidx, packed):
    mesh = plsc.VectorSubcoreMesh(core_axis_name="c", subcore_axis_name="s")
    k = pl.kernel(
        _gather_body,
        out_type=jax.ShapeDtypeStruct((BATCH, BLOCK), jnp.float32),
        mesh=mesh,
        scratch_types=[
            pltpu.VMEM((N_GCHUNK, GCHUNK), jnp.int32),
            pltpu.VMEM((B_PER_W, BLOCK), jnp.float32),
            pltpu.SemaphoreType.DMA,
        ],
    )
    return k(bidx, packed)


def _mlp_body(ub_ref, ib_ref, us_ref, is_ref, w1u_ref, w1v_ref, b1_ref,
              w2_ref, b2_ref, w3_ref, b3_ref, wo_ref, bo_ref, out_ref):
    tile = ub_ref.shape[0]
    grp = jax.lax.broadcasted_iota(jnp.int32, (tile, BLOCK), 1) // EMBED
    um = (grp == us_ref[...]).astype(jnp.float32)
    vm = (grp == is_ref[...]).astype(jnp.float32)
    x = (jnp.dot(ub_ref[...] * um, w1u_ref[...],
                 preferred_element_type=jnp.float32)
         + jnp.dot(ib_ref[...] * vm, w1v_ref[...],
                   preferred_element_type=jnp.float32)
         + b1_ref[...])
    x = jnp.maximum(x, 0.0)
    x = jnp.dot(x, w2_ref[...], preferred_element_type=jnp.float32) + b2_ref[...]
    x = jnp.maximum(x, 0.0)
    x = jnp.dot(x, w3_ref[...], preferred_element_type=jnp.float32) + b3_ref[...]
    x = jnp.maximum(x, 0.0)
    y = jnp.dot(x, wo_ref[...], preferred_element_type=jnp.float32) + bo_ref[...]
    out_ref[...] = 1.0 / (1.0 + jnp.exp(-y))


def _tc_mlp(ublocks, iblocks, usub, isub, W1, b1, W2, b2, W3, b3, Wo, bo,
            *, tile=2048):
    grid = BATCH // tile
    full = lambda shape: pl.BlockSpec(shape, lambda i: (0, 0))
    return pl.pallas_call(
        _mlp_body,
        grid=(grid,),
        in_specs=[
            pl.BlockSpec((tile, BLOCK), lambda i: (i, 0)),
            pl.BlockSpec((tile, BLOCK), lambda i: (i, 0)),
            pl.BlockSpec((tile, 1), lambda i: (i, 0)),
            pl.BlockSpec((tile, 1), lambda i: (i, 0)),
            full((BLOCK, 128)),
            full((BLOCK, 128)),
            full((1, 128)),
            full((128, 64)),
            full((1, 64)),
            full((64, 32)),
            full((1, 32)),
            full((32, 1)),
            full((1, 1)),
        ],
        out_specs=pl.BlockSpec((tile, 1), lambda i: (i, 0)),
        out_shape=jax.ShapeDtypeStruct((BATCH, 1), jnp.float32),
        compiler_params=pltpu.CompilerParams(
            dimension_semantics=("parallel",)),
    )(ublocks, iblocks, usub, isub,
      jnp.tile(W1[:EMBED], (PACK, 1)), jnp.tile(W1[EMBED:], (PACK, 1)),
      b1.reshape(1, -1), W2, b2.reshape(1, -1),
      W3, b3.reshape(1, -1), Wo, bo.reshape(1, -1))


def kernel(user_indices, item_indices, user_table, item_table,
           W1, b1, W2, b2, W3, b3, Wo, bo):
    uidx = user_indices.astype(jnp.int32)
    iidx = item_indices.astype(jnp.int32)
    ubidx = (128 * (uidx // CHUNK) + uidx % 128).reshape(IDX_ROWS, GCHUNK)
    ibidx = (128 * (iidx // CHUNK) + iidx % 128).reshape(IDX_ROWS, GCHUNK)
    upacked = _tc_pack(user_table.T)
    ublocks = _sc_gather(ubidx, upacked)
    ipacked = _tc_pack(item_table.T)
    iblocks = _sc_gather(ibidx, ipacked)
    usub = ((uidx // 128) % PACK).reshape(BATCH, 1)
    isub = ((iidx // 128) % PACK).reshape(BATCH, 1)
    return _tc_mlp(ublocks, iblocks, usub, isub,
                   W1, b1, W2, b2, W3, b3, Wo, bo)


# speedup vs baseline: 2.8586x; 1.3522x over previous
import jax
import jax.numpy as jnp
from jax import lax
from jax.experimental import pallas as pl
from jax.experimental.pallas import tpu as pltpu
from jax.experimental.pallas import tpu_sc as plsc

BATCH = 16384
EMBED = 32
ROWS = 1_000_000
BLOCK = 128
PACK = BLOCK // EMBED
CHUNK = 512
TPG = 16
PGRID = -(-ROWS // (CHUNK * TPG))
PROWS = PGRID * TPG * 128
NUM_CORES = 2
NUM_SUBCORES = 16
NUM_WORKERS = NUM_CORES * NUM_SUBCORES
B_PER_W = BATCH // NUM_WORKERS
GCHUNK = 128
N_GCHUNK = B_PER_W // GCHUNK
IDX_ROWS = BATCH // GCHUNK


def _pack_body(x_ref, out_ref):
    for tt in range(TPG):
        x4 = jnp.concatenate(
            [x_ref[:, CHUNK * tt + 128 * c:CHUNK * tt + 128 * c + 128]
             for c in range(PACK)], axis=0)
        out_ref[128 * tt:128 * tt + 128, :] = x4.T


def _tc_pack(table_t):
    return pl.pallas_call(
        _pack_body,
        grid=(PGRID,),
        in_specs=[pl.BlockSpec((EMBED, CHUNK * TPG), lambda i: (0, i))],
        out_specs=pl.BlockSpec((TPG * 128, BLOCK), lambda i: (i, 0)),
        out_shape=jax.ShapeDtypeStruct((PROWS, BLOCK), jnp.float32),
        compiler_params=pltpu.CompilerParams(
            dimension_semantics=("parallel",)),
    )(table_t)


def _gather_body(idx_hbm, t_hbm, out_hbm, idx_v, rows_v, sem):
    wid = lax.axis_index("s") * NUM_CORES + lax.axis_index("c")
    base = wid * B_PER_W
    irow = wid * N_GCHUNK
    pltpu.sync_copy(idx_hbm.at[pl.ds(irow, N_GCHUNK), :], idx_v)
    for j in range(N_GCHUNK):
        pltpu.async_copy(t_hbm.at[idx_v.at[j]],
                         rows_v.at[pl.ds(j * GCHUNK, GCHUNK), :], sem)
    for j in range(N_GCHUNK):
        pltpu.make_async_copy(t_hbm.at[idx_v.at[j]],
                              rows_v.at[pl.ds(j * GCHUNK, GCHUNK), :],
                              sem).wait()
    pltpu.sync_copy(rows_v, out_hbm.at[pl.ds(base, B_PER_W), :])


def _sc_gather(bidx, packed):
    mesh = plsc.VectorSubcoreMesh(core_axis_name="c", subcore_axis_name="s")
    k = pl.kernel(
        _gather_body,
        out_type=jax.ShapeDtypeStruct((BATCH, BLOCK), jnp.float32),
        mesh=mesh,
        scratch_types=[
            pltpu.VMEM((N_GCHUNK, GCHUNK), jnp.int32),
            pltpu.VMEM((B_PER_W, BLOCK), jnp.float32),
            pltpu.SemaphoreType.DMA,
        ],
    )
    return k(bidx, packed)


def _mlp_body(ub_ref, ib_ref, us_ref, is_ref, w1u_ref, w1v_ref, b1_ref,
              w2_ref, b2_ref, w3_ref, b3_ref, wo_ref, bo_ref, out_ref):
    tile = ub_ref.shape[0]
    grp = jax.lax.broadcasted_iota(jnp.int32, (tile, BLOCK), 1) // EMBED
    um = (grp == us_ref[...]).astype(jnp.float32)
    vm = (grp == is_ref[...]).astype(jnp.float32)
    x = (jnp.dot(ub_ref[...] * um, w1u_ref[...],
                 preferred_element_type=jnp.float32)
         + jnp.dot(ib_ref[...] * vm, w1v_ref[...],
                   preferred_element_type=jnp.float32)
         + b1_ref[...])
    x = jnp.maximum(x, 0.0)
    x = jnp.dot(x, w2_ref[...], preferred_element_type=jnp.float32) + b2_ref[...]
    x = jnp.maximum(x, 0.0)
    x = jnp.dot(x, w3_ref[...], preferred_element_type=jnp.float32) + b3_ref[...]
    x = jnp.maximum(x, 0.0)
    y = jnp.dot(x, wo_ref[...], preferred_element_type=jnp.float32) + bo_ref[...]
    out_ref[...] = 1.0 / (1.0 + jnp.exp(-y))


def _tc_mlp(ublocks, iblocks, usub, isub, W1, b1, W2, b2, W3, b3, Wo, bo,
            *, tile=2048):
    grid = BATCH // tile
    full = lambda shape: pl.BlockSpec(shape, lambda i: (0, 0))
    return pl.pallas_call(
        _mlp_body,
        grid=(grid,),
        in_specs=[
            pl.BlockSpec((tile, BLOCK), lambda i: (i, 0)),
            pl.BlockSpec((tile, BLOCK), lambda i: (i, 0)),
            pl.BlockSpec((tile, 1), lambda i: (i, 0)),
            pl.BlockSpec((tile, 1), lambda i: (i, 0)),
            full((BLOCK, 128)),
            full((BLOCK, 128)),
            full((1, 128)),
            full((128, 64)),
            full((1, 64)),
            full((64, 32)),
            full((1, 32)),
            full((32, 1)),
            full((1, 1)),
        ],
        out_specs=pl.BlockSpec((tile, 1), lambda i: (i, 0)),
        out_shape=jax.ShapeDtypeStruct((BATCH, 1), jnp.float32),
        compiler_params=pltpu.CompilerParams(
            dimension_semantics=("parallel",)),
    )(ublocks, iblocks, usub, isub,
      jnp.tile(W1[:EMBED], (PACK, 1)), jnp.tile(W1[EMBED:], (PACK, 1)),
      b1.reshape(1, -1), W2, b2.reshape(1, -1),
      W3, b3.reshape(1, -1), Wo, bo.reshape(1, -1))


def kernel(user_indices, item_indices, user_table, item_table,
           W1, b1, W2, b2, W3, b3, Wo, bo):
    uidx = user_indices.astype(jnp.int32)
    iidx = item_indices.astype(jnp.int32)
    ubidx = (128 * (uidx // CHUNK) + uidx % 128).reshape(IDX_ROWS, GCHUNK)
    ibidx = (128 * (iidx // CHUNK) + iidx % 128).reshape(IDX_ROWS, GCHUNK)
    upacked = _tc_pack(user_table.T)
    ublocks = _sc_gather(ubidx, upacked)
    ipacked = _tc_pack(item_table.T)
    iblocks = _sc_gather(ibidx, ipacked)
    usub = ((uidx // 128) % PACK).reshape(BATCH, 1)
    isub = ((iidx // 128) % PACK).reshape(BATCH, 1)
    return _tc_mlp(ublocks, iblocks, usub, isub,
                   W1, b1, W2, b2, W3, b3, Wo, bo)


# speedup vs baseline: 3.6386x; 1.2729x over previous
import jax
import jax.numpy as jnp
from jax import lax
from jax.experimental import pallas as pl
from jax.experimental.pallas import tpu as pltpu
from jax.experimental.pallas import tpu_sc as plsc

BATCH = 16384
EMBED = 32
ROWS = 1_000_000
BLOCK = 128
PACK = BLOCK // EMBED
CHUNK = 512
TPG = 32
PGRID = -(-ROWS // (CHUNK * TPG))
PROWS = PGRID * TPG * 128
NUM_CORES = 2
NUM_SUBCORES = 16
NUM_WORKERS = NUM_CORES * NUM_SUBCORES
B_PER_W = BATCH // NUM_WORKERS
GCHUNK = 128
N_GCHUNK = B_PER_W // GCHUNK
IDX_ROWS = BATCH // GCHUNK


def _pack_body(x_ref, out_ref):
    for tt in range(TPG):
        x4 = jnp.concatenate(
            [x_ref[:, CHUNK * tt + 128 * c:CHUNK * tt + 128 * c + 128]
             for c in range(PACK)], axis=0)
        out_ref[128 * tt:128 * tt + 128, :] = x4.T


def _tc_pack(table_t):
    return pl.pallas_call(
        _pack_body,
        grid=(PGRID,),
        in_specs=[pl.BlockSpec((EMBED, CHUNK * TPG), lambda i: (0, i))],
        out_specs=pl.BlockSpec((TPG * 128, BLOCK), lambda i: (i, 0)),
        out_shape=jax.ShapeDtypeStruct((PROWS, BLOCK), jnp.float32),
        compiler_params=pltpu.CompilerParams(
            dimension_semantics=("parallel",)),
    )(table_t)


def _gather_body(idx_hbm, t_hbm, out_hbm, idx_v, rows_v, sem):
    wid = lax.axis_index("s") * NUM_CORES + lax.axis_index("c")
    base = wid * B_PER_W
    irow = wid * N_GCHUNK
    pltpu.sync_copy(idx_hbm.at[pl.ds(irow, N_GCHUNK), :], idx_v)
    for j in range(N_GCHUNK):
        pltpu.async_copy(t_hbm.at[idx_v.at[j]],
                         rows_v.at[pl.ds(j * GCHUNK, GCHUNK), :], sem)
    for j in range(N_GCHUNK):
        pltpu.make_async_copy(t_hbm.at[idx_v.at[j]],
                              rows_v.at[pl.ds(j * GCHUNK, GCHUNK), :],
                              sem).wait()
    pltpu.sync_copy(rows_v, out_hbm.at[pl.ds(base, B_PER_W), :])


def _sc_gather(bidx, packed):
    mesh = plsc.VectorSubcoreMesh(core_axis_name="c", subcore_axis_name="s")
    k = pl.kernel(
        _gather_body,
        out_type=jax.ShapeDtypeStruct((BATCH, BLOCK), jnp.float32),
        mesh=mesh,
        scratch_types=[
            pltpu.VMEM((N_GCHUNK, GCHUNK), jnp.int32),
            pltpu.VMEM((B_PER_W, BLOCK), jnp.float32),
            pltpu.SemaphoreType.DMA,
        ],
    )
    return k(bidx, packed)


def _mlp_body(ub_ref, ib_ref, us_ref, is_ref, w1u_ref, w1v_ref, b1_ref,
              w2_ref, b2_ref, w3_ref, b3_ref, wo_ref, bo_ref, out_ref):
    tile = ub_ref.shape[0]
    grp = jax.lax.broadcasted_iota(jnp.int32, (tile, BLOCK), 1) // EMBED
    um = (grp == us_ref[...]).astype(jnp.float32)
    vm = (grp == is_ref[...]).astype(jnp.float32)
    x = (jnp.dot(ub_ref[...] * um, w1u_ref[...],
                 preferred_element_type=jnp.float32)
         + jnp.dot(ib_ref[...] * vm, w1v_ref[...],
                   preferred_element_type=jnp.float32)
         + b1_ref[...])
    x = jnp.maximum(x, 0.0)
    x = jnp.dot(x, w2_ref[...], preferred_element_type=jnp.float32) + b2_ref[...]
    x = jnp.maximum(x, 0.0)
    x = jnp.dot(x, w3_ref[...], preferred_element_type=jnp.float32) + b3_ref[...]
    x = jnp.maximum(x, 0.0)
    y = jnp.dot(x, wo_ref[...], preferred_element_type=jnp.float32) + bo_ref[...]
    out_ref[...] = 1.0 / (1.0 + jnp.exp(-y))


def _tc_mlp(ublocks, iblocks, usub, isub, W1, b1, W2, b2, W3, b3, Wo, bo,
            *, tile=2048):
    grid = BATCH // tile
    full = lambda shape: pl.BlockSpec(shape, lambda i: (0, 0))
    return pl.pallas_call(
        _mlp_body,
        grid=(grid,),
        in_specs=[
            pl.BlockSpec((tile, BLOCK), lambda i: (i, 0)),
            pl.BlockSpec((tile, BLOCK), lambda i: (i, 0)),
            pl.BlockSpec((tile, 1), lambda i: (i, 0)),
            pl.BlockSpec((tile, 1), lambda i: (i, 0)),
            full((BLOCK, 128)),
            full((BLOCK, 128)),
            full((1, 128)),
            full((128, 64)),
            full((1, 64)),
            full((64, 32)),
            full((1, 32)),
            full((32, 1)),
            full((1, 1)),
        ],
        out_specs=pl.BlockSpec((tile, 1), lambda i: (i, 0)),
        out_shape=jax.ShapeDtypeStruct((BATCH, 1), jnp.float32),
        compiler_params=pltpu.CompilerParams(
            dimension_semantics=("parallel",)),
    )(ublocks, iblocks, usub, isub,
      jnp.tile(W1[:EMBED], (PACK, 1)), jnp.tile(W1[EMBED:], (PACK, 1)),
      b1.reshape(1, -1), W2, b2.reshape(1, -1),
      W3, b3.reshape(1, -1), Wo, bo.reshape(1, -1))


def kernel(user_indices, item_indices, user_table, item_table,
           W1, b1, W2, b2, W3, b3, Wo, bo):
    uidx = user_indices.astype(jnp.int32)
    iidx = item_indices.astype(jnp.int32)
    ubidx = (128 * (uidx // CHUNK) + uidx % 128).reshape(IDX_ROWS, GCHUNK)
    ibidx = (128 * (iidx // CHUNK) + iidx % 128).reshape(IDX_ROWS, GCHUNK)
    upacked = _tc_pack(user_table.T)
    ublocks = _sc_gather(ubidx, upacked)
    ipacked = _tc_pack(item_table.T)
    iblocks = _sc_gather(ibidx, ipacked)
    usub = ((uidx // 128) % PACK).reshape(BATCH, 1)
    isub = ((iidx // 128) % PACK).reshape(BATCH, 1)
    return _tc_mlp(ublocks, iblocks, usub, isub,
                   W1, b1, W2, b2, W3, b3, Wo, bo)


# speedup vs baseline: 4.0915x; 1.1245x over previous
import jax
import jax.numpy as jnp
from jax import lax
from jax.experimental import pallas as pl
from jax.experimental.pallas import tpu as pltpu
from jax.experimental.pallas import tpu_sc as plsc

BATCH = 16384
EMBED = 32
ROWS = 1_000_000
BLOCK = 128
PACK = BLOCK // EMBED
CHUNK = 512
TPG = 64
PGRID = -(-ROWS // (CHUNK * TPG))
PROWS = PGRID * TPG * 128
NUM_CORES = 2
NUM_SUBCORES = 16
NUM_WORKERS = NUM_CORES * NUM_SUBCORES
B_PER_W = BATCH // NUM_WORKERS
GCHUNK = 128
N_GCHUNK = B_PER_W // GCHUNK
IDX_ROWS = BATCH // GCHUNK


def _pack_body(x_ref, out_ref):
    for tt in range(TPG):
        x4 = jnp.concatenate(
            [x_ref[:, CHUNK * tt + 128 * c:CHUNK * tt + 128 * c + 128]
             for c in range(PACK)], axis=0)
        out_ref[128 * tt:128 * tt + 128, :] = x4.T


def _tc_pack(table_t):
    return pl.pallas_call(
        _pack_body,
        grid=(PGRID,),
        in_specs=[pl.BlockSpec((EMBED, CHUNK * TPG), lambda i: (0, i))],
        out_specs=pl.BlockSpec((TPG * 128, BLOCK), lambda i: (i, 0)),
        out_shape=jax.ShapeDtypeStruct((PROWS, BLOCK), jnp.float32),
        compiler_params=pltpu.CompilerParams(
            dimension_semantics=("parallel",)),
    )(table_t)


def _gather_body(idx_hbm, t_hbm, out_hbm, idx_v, rows_v, sem):
    wid = lax.axis_index("s") * NUM_CORES + lax.axis_index("c")
    base = wid * B_PER_W
    irow = wid * N_GCHUNK
    pltpu.sync_copy(idx_hbm.at[pl.ds(irow, N_GCHUNK), :], idx_v)
    for j in range(N_GCHUNK):
        pltpu.async_copy(t_hbm.at[idx_v.at[j]],
                         rows_v.at[pl.ds(j * GCHUNK, GCHUNK), :], sem)
    for j in range(N_GCHUNK):
        pltpu.make_async_copy(t_hbm.at[idx_v.at[j]],
                              rows_v.at[pl.ds(j * GCHUNK, GCHUNK), :],
                              sem).wait()
    pltpu.sync_copy(rows_v, out_hbm.at[pl.ds(base, B_PER_W), :])


def _sc_gather(bidx, packed):
    mesh = plsc.VectorSubcoreMesh(core_axis_name="c", subcore_axis_name="s")
    k = pl.kernel(
        _gather_body,
        out_type=jax.ShapeDtypeStruct((BATCH, BLOCK), jnp.float32),
        mesh=mesh,
        scratch_types=[
            pltpu.VMEM((N_GCHUNK, GCHUNK), jnp.int32),
            pltpu.VMEM((B_PER_W, BLOCK), jnp.float32),
            pltpu.SemaphoreType.DMA,
        ],
    )
    return k(bidx, packed)


def _mlp_body(ub_ref, ib_ref, us_ref, is_ref, w1u_ref, w1v_ref, b1_ref,
              w2_ref, b2_ref, w3_ref, b3_ref, wo_ref, bo_ref, out_ref):
    tile = ub_ref.shape[0]
    grp = jax.lax.broadcasted_iota(jnp.int32, (tile, BLOCK), 1) // EMBED
    um = (grp == us_ref[...]).astype(jnp.float32)
    vm = (grp == is_ref[...]).astype(jnp.float32)
    x = (jnp.dot(ub_ref[...] * um, w1u_ref[...],
                 preferred_element_type=jnp.float32)
         + jnp.dot(ib_ref[...] * vm, w1v_ref[...],
                   preferred_element_type=jnp.float32)
         + b1_ref[...])
    x = jnp.maximum(x, 0.0)
    x = jnp.dot(x, w2_ref[...], preferred_element_type=jnp.float32) + b2_ref[...]
    x = jnp.maximum(x, 0.0)
    x = jnp.dot(x, w3_ref[...], preferred_element_type=jnp.float32) + b3_ref[...]
    x = jnp.maximum(x, 0.0)
    y = jnp.dot(x, wo_ref[...], preferred_element_type=jnp.float32) + bo_ref[...]
    out_ref[...] = 1.0 / (1.0 + jnp.exp(-y))


def _tc_mlp(ublocks, iblocks, usub, isub, W1, b1, W2, b2, W3, b3, Wo, bo,
            *, tile=2048):
    grid = BATCH // tile
    full = lambda shape: pl.BlockSpec(shape, lambda i: (0, 0))
    return pl.pallas_call(
        _mlp_body,
        grid=(grid,),
        in_specs=[
            pl.BlockSpec((tile, BLOCK), lambda i: (i, 0)),
            pl.BlockSpec((tile, BLOCK), lambda i: (i, 0)),
            pl.BlockSpec((tile, 1), lambda i: (i, 0)),
            pl.BlockSpec((tile, 1), lambda i: (i, 0)),
            full((BLOCK, 128)),
            full((BLOCK, 128)),
            full((1, 128)),
            full((128, 64)),
            full((1, 64)),
            full((64, 32)),
            full((1, 32)),
            full((32, 1)),
            full((1, 1)),
        ],
        out_specs=pl.BlockSpec((tile, 1), lambda i: (i, 0)),
        out_shape=jax.ShapeDtypeStruct((BATCH, 1), jnp.float32),
        compiler_params=pltpu.CompilerParams(
            dimension_semantics=("parallel",)),
    )(ublocks, iblocks, usub, isub,
      jnp.tile(W1[:EMBED], (PACK, 1)), jnp.tile(W1[EMBED:], (PACK, 1)),
      b1.reshape(1, -1), W2, b2.reshape(1, -1),
      W3, b3.reshape(1, -1), Wo, bo.reshape(1, -1))


def kernel(user_indices, item_indices, user_table, item_table,
           W1, b1, W2, b2, W3, b3, Wo, bo):
    uidx = user_indices.astype(jnp.int32)
    iidx = item_indices.astype(jnp.int32)
    ubidx = (128 * (uidx // CHUNK) + uidx % 128).reshape(IDX_ROWS, GCHUNK)
    ibidx = (128 * (iidx // CHUNK) + iidx % 128).reshape(IDX_ROWS, GCHUNK)
    upacked = _tc_pack(user_table.T)
    ublocks = _sc_gather(ubidx, upacked)
    ipacked = _tc_pack(item_table.T)
    iblocks = _sc_gather(ibidx, ipacked)
    usub = ((uidx // 128) % PACK).reshape(BATCH, 1)
    isub = ((iidx // 128) % PACK).reshape(BATCH, 1)
    return _tc_mlp(ublocks, iblocks, usub, isub,
                   W1, b1, W2, b2, W3, b3, Wo, bo)


# speedup vs baseline: 4.1439x; 1.0128x over previous
import jax
import jax.numpy as jnp
from jax import lax
from jax.experimental import pallas as pl
from jax.experimental.pallas import tpu as pltpu
from jax.experimental.pallas import tpu_sc as plsc

BATCH = 16384
EMBED = 32
ROWS = 1_000_000
BLOCK = 128
PACK = BLOCK // EMBED
CHUNK = 512
TPG = 128
PGRID = -(-ROWS // (CHUNK * TPG))
PROWS = PGRID * TPG * 128
NUM_CORES = 2
NUM_SUBCORES = 16
NUM_WORKERS = NUM_CORES * NUM_SUBCORES
B_PER_W = BATCH // NUM_WORKERS
GCHUNK = 128
N_GCHUNK = B_PER_W // GCHUNK
IDX_ROWS = BATCH // GCHUNK


def _pack_body(x_ref, out_ref):
    for tt in range(TPG):
        x4 = jnp.concatenate(
            [x_ref[:, CHUNK * tt + 128 * c:CHUNK * tt + 128 * c + 128]
             for c in range(PACK)], axis=0)
        out_ref[128 * tt:128 * tt + 128, :] = x4.T


def _tc_pack(table_t):
    return pl.pallas_call(
        _pack_body,
        grid=(PGRID,),
        in_specs=[pl.BlockSpec((EMBED, CHUNK * TPG), lambda i: (0, i))],
        out_specs=pl.BlockSpec((TPG * 128, BLOCK), lambda i: (i, 0)),
        out_shape=jax.ShapeDtypeStruct((PROWS, BLOCK), jnp.float32),
        compiler_params=pltpu.CompilerParams(
            dimension_semantics=("parallel",)),
    )(table_t)


def _gather_body(idx_hbm, t_hbm, out_hbm, idx_v, rows_v, sem):
    wid = lax.axis_index("s") * NUM_CORES + lax.axis_index("c")
    base = wid * B_PER_W
    irow = wid * N_GCHUNK
    pltpu.sync_copy(idx_hbm.at[pl.ds(irow, N_GCHUNK), :], idx_v)
    for j in range(N_GCHUNK):
        pltpu.async_copy(t_hbm.at[idx_v.at[j]],
                         rows_v.at[pl.ds(j * GCHUNK, GCHUNK), :], sem)
    for j in range(N_GCHUNK):
        pltpu.make_async_copy(t_hbm.at[idx_v.at[j]],
                              rows_v.at[pl.ds(j * GCHUNK, GCHUNK), :],
                              sem).wait()
    pltpu.sync_copy(rows_v, out_hbm.at[pl.ds(base, B_PER_W), :])


def _sc_gather(bidx, packed):
    mesh = plsc.VectorSubcoreMesh(core_axis_name="c", subcore_axis_name="s")
    k = pl.kernel(
        _gather_body,
        out_type=jax.ShapeDtypeStruct((BATCH, BLOCK), jnp.float32),
        mesh=mesh,
        scratch_types=[
            pltpu.VMEM((N_GCHUNK, GCHUNK), jnp.int32),
            pltpu.VMEM((B_PER_W, BLOCK), jnp.float32),
            pltpu.SemaphoreType.DMA,
        ],
    )
    return k(bidx, packed)


def _mlp_body(ub_ref, ib_ref, us_ref, is_ref, w1u_ref, w1v_ref, b1_ref,
              w2_ref, b2_ref, w3_ref, b3_ref, wo_ref, bo_ref, out_ref):
    tile = ub_ref.shape[0]
    grp = jax.lax.broadcasted_iota(jnp.int32, (tile, BLOCK), 1) // EMBED
    um = (grp == us_ref[...]).astype(jnp.float32)
    vm = (grp == is_ref[...]).astype(jnp.float32)
    x = (jnp.dot(ub_ref[...] * um, w1u_ref[...],
                 preferred_element_type=jnp.float32)
         + jnp.dot(ib_ref[...] * vm, w1v_ref[...],
                   preferred_element_type=jnp.float32)
         + b1_ref[...])
    x = jnp.maximum(x, 0.0)
    x = jnp.dot(x, w2_ref[...], preferred_element_type=jnp.float32) + b2_ref[...]
    x = jnp.maximum(x, 0.0)
    x = jnp.dot(x, w3_ref[...], preferred_element_type=jnp.float32) + b3_ref[...]
    x = jnp.maximum(x, 0.0)
    y = jnp.dot(x, wo_ref[...], preferred_element_type=jnp.float32) + bo_ref[...]
    out_ref[...] = 1.0 / (1.0 + jnp.exp(-y))


def _tc_mlp(ublocks, iblocks, usub, isub, W1, b1, W2, b2, W3, b3, Wo, bo,
            *, tile=2048):
    grid = BATCH // tile
    full = lambda shape: pl.BlockSpec(shape, lambda i: (0, 0))
    return pl.pallas_call(
        _mlp_body,
        grid=(grid,),
        in_specs=[
            pl.BlockSpec((tile, BLOCK), lambda i: (i, 0)),
            pl.BlockSpec((tile, BLOCK), lambda i: (i, 0)),
            pl.BlockSpec((tile, 1), lambda i: (i, 0)),
            pl.BlockSpec((tile, 1), lambda i: (i, 0)),
            full((BLOCK, 128)),
            full((BLOCK, 128)),
            full((1, 128)),
            full((128, 64)),
            full((1, 64)),
            full((64, 32)),
            full((1, 32)),
            full((32, 1)),
            full((1, 1)),
        ],
        out_specs=pl.BlockSpec((tile, 1), lambda i: (i, 0)),
        out_shape=jax.ShapeDtypeStruct((BATCH, 1), jnp.float32),
        compiler_params=pltpu.CompilerParams(
            dimension_semantics=("parallel",)),
    )(ublocks, iblocks, usub, isub,
      jnp.tile(W1[:EMBED], (PACK, 1)), jnp.tile(W1[EMBED:], (PACK, 1)),
      b1.reshape(1, -1), W2, b2.reshape(1, -1),
      W3, b3.reshape(1, -1), Wo, bo.reshape(1, -1))


def kernel(user_indices, item_indices, user_table, item_table,
           W1, b1, W2, b2, W3, b3, Wo, bo):
    uidx = user_indices.astype(jnp.int32)
    iidx = item_indices.astype(jnp.int32)
    ubidx = (128 * (uidx // CHUNK) + uidx % 128).reshape(IDX_ROWS, GCHUNK)
    ibidx = (128 * (iidx // CHUNK) + iidx % 128).reshape(IDX_ROWS, GCHUNK)
    upacked = _tc_pack(user_table.T)
    ublocks = _sc_gather(ubidx, upacked)
    ipacked = _tc_pack(item_table.T)
    iblocks = _sc_gather(ibidx, ipacked)
    usub = ((uidx // 128) % PACK).reshape(BATCH, 1)
    isub = ((iidx // 128) % PACK).reshape(BATCH, 1)
    return _tc_mlp(ublocks, iblocks, usub, isub,
                   W1, b1, W2, b2, W3, b3, Wo, bo)


# speedup vs baseline: 4.1502x; 1.0015x over previous
import jax
import jax.numpy as jnp
from jax import lax
from jax.experimental import pallas as pl
from jax.experimental.pallas import tpu as pltpu
from jax.experimental.pallas import tpu_sc as plsc

BATCH = 16384
EMBED = 32
ROWS = 1_000_000
BLOCK = 128
PACK = BLOCK // EMBED
CHUNK = 512
TPG = 128
PGRID = -(-ROWS // (CHUNK * TPG))
PROWS = PGRID * TPG * 128
NUM_CORES = 2
NUM_SUBCORES = 16
NUM_WORKERS = NUM_CORES * NUM_SUBCORES
B_PER_W = BATCH // NUM_WORKERS
GCHUNK = 128
N_GCHUNK = B_PER_W // GCHUNK
IDX_ROWS = BATCH // GCHUNK


def _pack_body(x_ref, out_ref):
    for tt in range(TPG):
        x4 = jnp.concatenate(
            [x_ref[:, CHUNK * tt + 128 * c:CHUNK * tt + 128 * c + 128]
             for c in range(PACK)], axis=0)
        out_ref[128 * tt:128 * tt + 128, :] = x4.T


def _tc_pack(table_t):
    return pl.pallas_call(
        _pack_body,
        grid=(PGRID,),
        in_specs=[pl.BlockSpec((EMBED, CHUNK * TPG), lambda i: (0, i))],
        out_specs=pl.BlockSpec((TPG * 128, BLOCK), lambda i: (i, 0)),
        out_shape=jax.ShapeDtypeStruct((PROWS, BLOCK), jnp.float32),
        compiler_params=pltpu.CompilerParams(
            dimension_semantics=("parallel",)),
    )(table_t)


def _gather_body(idx_hbm, t_hbm, out_hbm, idx_v, rows_v, sem):
    wid = lax.axis_index("s") * NUM_CORES + lax.axis_index("c")
    base = wid * B_PER_W
    irow = wid * N_GCHUNK
    pltpu.sync_copy(idx_hbm.at[pl.ds(irow, N_GCHUNK), :], idx_v)
    for j in range(N_GCHUNK):
        pltpu.async_copy(t_hbm.at[idx_v.at[j]],
                         rows_v.at[pl.ds(j * GCHUNK, GCHUNK), :], sem)
    for j in range(N_GCHUNK):
        pltpu.make_async_copy(t_hbm.at[idx_v.at[j]],
                              rows_v.at[pl.ds(j * GCHUNK, GCHUNK), :],
                              sem).wait()
    pltpu.sync_copy(rows_v, out_hbm.at[pl.ds(base, B_PER_W), :])


def _sc_gather(bidx, packed):
    mesh = plsc.VectorSubcoreMesh(core_axis_name="c", subcore_axis_name="s")
    k = pl.kernel(
        _gather_body,
        out_type=jax.ShapeDtypeStruct((BATCH, BLOCK), jnp.float32),
        mesh=mesh,
        scratch_types=[
            pltpu.VMEM((N_GCHUNK, GCHUNK), jnp.int32),
            pltpu.VMEM((B_PER_W, BLOCK), jnp.float32),
            pltpu.SemaphoreType.DMA,
        ],
    )
    return k(bidx, packed)


def _mlp_body(ub_ref, ib_ref, us_ref, is_ref, w1u_ref, w1v_ref, b1_ref,
              w2_ref, b2_ref, w3_ref, b3_ref, wo_ref, bo_ref, out_ref):
    tile = ub_ref.shape[0]
    grp = jax.lax.broadcasted_iota(jnp.int32, (tile, BLOCK), 1) // EMBED
    um = (grp == us_ref[...]).astype(jnp.float32)
    vm = (grp == is_ref[...]).astype(jnp.float32)
    x = (jnp.dot(ub_ref[...] * um, w1u_ref[...],
                 preferred_element_type=jnp.float32)
         + jnp.dot(ib_ref[...] * vm, w1v_ref[...],
                   preferred_element_type=jnp.float32)
         + b1_ref[...])
    x = jnp.maximum(x, 0.0)
    x = jnp.dot(x, w2_ref[...], preferred_element_type=jnp.float32) + b2_ref[...]
    x = jnp.maximum(x, 0.0)
    x = jnp.dot(x, w3_ref[...], preferred_element_type=jnp.float32) + b3_ref[...]
    x = jnp.maximum(x, 0.0)
    y = jnp.dot(x, wo_ref[...], preferred_element_type=jnp.float32) + bo_ref[...]
    out_ref[...] = 1.0 / (1.0 + jnp.exp(-y))


def _tc_mlp(ublocks, iblocks, usub, isub, W1, b1, W2, b2, W3, b3, Wo, bo,
            *, tile=8192):
    grid = BATCH // tile
    full = lambda shape: pl.BlockSpec(shape, lambda i: (0, 0))
    return pl.pallas_call(
        _mlp_body,
        grid=(grid,),
        in_specs=[
            pl.BlockSpec((tile, BLOCK), lambda i: (i, 0)),
            pl.BlockSpec((tile, BLOCK), lambda i: (i, 0)),
            pl.BlockSpec((tile, 1), lambda i: (i, 0)),
            pl.BlockSpec((tile, 1), lambda i: (i, 0)),
            full((BLOCK, 128)),
            full((BLOCK, 128)),
            full((1, 128)),
            full((128, 64)),
            full((1, 64)),
            full((64, 32)),
            full((1, 32)),
            full((32, 1)),
            full((1, 1)),
        ],
        out_specs=pl.BlockSpec((tile, 1), lambda i: (i, 0)),
        out_shape=jax.ShapeDtypeStruct((BATCH, 1), jnp.float32),
        compiler_params=pltpu.CompilerParams(
            dimension_semantics=("parallel",)),
    )(ublocks, iblocks, usub, isub,
      jnp.tile(W1[:EMBED], (PACK, 1)), jnp.tile(W1[EMBED:], (PACK, 1)),
      b1.reshape(1, -1), W2, b2.reshape(1, -1),
      W3, b3.reshape(1, -1), Wo, bo.reshape(1, -1))


def kernel(user_indices, item_indices, user_table, item_table,
           W1, b1, W2, b2, W3, b3, Wo, bo):
    uidx = user_indices.astype(jnp.int32)
    iidx = item_indices.astype(jnp.int32)
    ubidx = (128 * (uidx // CHUNK) + uidx % 128).reshape(IDX_ROWS, GCHUNK)
    ibidx = (128 * (iidx // CHUNK) + iidx % 128).reshape(IDX_ROWS, GCHUNK)
    upacked = _tc_pack(user_table.T)
    ublocks = _sc_gather(ubidx, upacked)
    ipacked = _tc_pack(item_table.T)
    iblocks = _sc_gather(ibidx, ipacked)
    usub = ((uidx // 128) % PACK).reshape(BATCH, 1)
    isub = ((iidx // 128) % PACK).reshape(BATCH, 1)
    return _tc_mlp(ublocks, iblocks, usub, isub,
                   W1, b1, W2, b2, W3, b3, Wo, bo)
